# baseline (speedup 1.0000x reference)
import functools

import jax
import jax.numpy as jnp
from jax import lax
from jax.experimental import pallas as pl
from jax.experimental.pallas import tpu as pltpu
from jax.experimental.pallas import tpu_sc as plsc

N = 50000
NP = 50176
RP = NP // 16
E = 800000
EP = 819200
F1 = 16
HID = 256
MID = 128
BN = 1024
BR = BN // 16
GN = NP // BN


def _k1_body(degp_ref, dinv_ref):
    deg = jnp.sum(degp_ref[...], axis=0) + 1.0
    dinv_ref[...] = 1.0 / jnp.sqrt(deg)


def _tc_k1(degp):
    return pl.pallas_call(
        _k1_body,
        grid=(NPP // BN,),
        in_specs=[pl.BlockSpec((NWK, BN), lambda i: (0, i))],
        out_specs=pl.BlockSpec((BN,), lambda i: (i,)),
        out_shape=jax.ShapeDtypeStruct((NPP,), jnp.float32),
    )(degp)


def _k1g_body(x_ref, dinv_ref, g1_ref):
    dflat = jnp.max(dinv_ref[...], axis=1)
    g1_ref[...] = x_ref[...] * dflat[:, None]


def _tc_k1g(x, dinv8):
    return pl.pallas_call(
        _k1g_body,
        grid=(GN,),
        in_specs=[
            pl.BlockSpec((BN, F1), lambda i: (i, 0)),
            pl.BlockSpec((BN, 8), lambda i: (i, 0)),
        ],
        out_specs=pl.BlockSpec((BN, F1), lambda i: (i, 0)),
        out_shape=jax.ShapeDtypeStruct((NP, F1), jnp.float32),
    )(x, dinv8)


def _k2_body(s1_ref, g1_ref, dinv_ref, w1_ref, b1_ref, w2_ref, *g2_refs):
    dflat = jnp.max(dinv_ref[...], axis=1)
    z1 = (s1_ref[0] + s1_ref[1] + g1_ref[...]) * dflat[:, None]
    h1 = jnp.dot(z1, w1_ref[...], preferred_element_type=jnp.float32)
    h1 = jnp.maximum(h1 + b1_ref[...], 0.0)
    m2 = jnp.dot(h1, w2_ref[...], preferred_element_type=jnp.float32)
    g2 = m2 * dflat[:, None]
    for d in range(8):
        g2_refs[d][...] = g2[:, d * 16:(d + 1) * 16]


def _tc_k2(s1p, g1, dinv8, w1p, b1, w2):
    return pl.pallas_call(
        _k2_body,
        grid=(GN,),
        in_specs=[
            pl.BlockSpec((2, BN, F1), lambda i: (0, i, 0)),
            pl.BlockSpec((BN, F1), lambda i: (i, 0)),
            pl.BlockSpec((BN, 8), lambda i: (i, 0)),
            pl.BlockSpec((F1, HID), lambda i: (0, 0)),
            pl.BlockSpec((1, HID), lambda i: (0, 0)),
            pl.BlockSpec((HID, MID), lambda i: (0, 0)),
        ],
        out_specs=[pl.BlockSpec((BN, 16), lambda i: (i, 0))] * 8,
        out_shape=[jax.ShapeDtypeStruct((NP, 16), jnp.float32)] * 8,
    )(s1p, g1, dinv8, w1p, b1, w2)


def _k3_body(*refs):
    srefs = refs[0:8]
    grefs = refs[8:16]
    dinv_ref, b2_ref, w3_ref, g3_ref = refs[16:]
    dflat = jnp.max(dinv_ref[...], axis=1)
    cols = [srefs[d][0] + srefs[d][1] + grefs[d][...] for d in range(8)]
    z2 = jnp.concatenate(cols, axis=1) * dflat[:, None]
    h2 = jnp.maximum(z2 + b2_ref[...], 0.0)
    m3 = jnp.sum(h2 * w3_ref[...], axis=1)
    g3_ref[...] = (m3 * dflat)[:, None] * jnp.ones((1, 8), jnp.float32)


def _tc_k3(s2p, g2s, dinv8, b2, w3):
    return pl.pallas_call(
        _k3_body,
        grid=(GN,),
        in_specs=(
            [pl.BlockSpec((2, BN, 16), lambda i: (0, i, 0))] * 8
            + [pl.BlockSpec((BN, 16), lambda i: (i, 0))] * 8
            + [
                pl.BlockSpec((BN, 8), lambda i: (i, 0)),
                pl.BlockSpec((1, MID), lambda i: (0, 0)),
                pl.BlockSpec((1, MID), lambda i: (0, 0)),
            ]
        ),
        out_specs=pl.BlockSpec((BN, 8), lambda i: (i, 0)),
        out_shape=jax.ShapeDtypeStruct((NP, 8), jnp.float32),
    )(*s2p, *g2s, dinv8, b2, w3)


def _k4_body(s3_ref, g3_ref, dinv_ref, b3_ref, out_ref):
    s3 = jnp.sum(s3_ref[...], axis=0)
    out_ref[...] = (s3 + g3_ref[...]) * dinv_ref[...] + b3_ref[...]


def _tc_k4(s3p, g3f, dinv1, b3b):
    return pl.pallas_call(
        _k4_body,
        grid=(NPP // BN,),
        in_specs=[
            pl.BlockSpec((NWK, BN), lambda i: (0, i)),
            pl.BlockSpec((BN,), lambda i: (i,)),
            pl.BlockSpec((BN,), lambda i: (i,)),
            pl.BlockSpec((BN,), lambda i: (i,)),
        ],
        out_specs=pl.BlockSpec((BN,), lambda i: (i,)),
        out_shape=jax.ShapeDtypeStruct((NPP,), jnp.float32),
    )(s3p, g3f, dinv1, b3b)


NC, NS, NWK = 2, 16, 32
NB, KB = 400, 64
RPP = 3200
NPP = RPP * 16
RT = RPP // NS
RTN = NP // NS
IDR, IDC = 25, 128

_MESH = dict(core_axis_name="c", subcore_axis_name="s")


def _zero_rows16(ref, n):
    def body(i, carry):
        ref[i] = jnp.zeros((16,), jnp.float32)
        return carry
    lax.fori_loop(0, n, body, None)


def _zero_rows32(ref, n):
    def body(i, carry):
        ref[i, pl.ds(0, 16)] = jnp.zeros((16,), jnp.float32)
        ref[i, pl.ds(16, 16)] = jnp.zeros((16,), jnp.float32)
        return carry
    lax.fori_loop(0, n, body, None)


def _sc_deg(dst3):
    @functools.partial(
        pl.kernel,
        out_type=jax.ShapeDtypeStruct((NWK * NPP,), jnp.float32),
        mesh=plsc.VectorSubcoreMesh(**_MESH),
        compiler_params=pltpu.CompilerParams(needs_layout_passes=False, use_tc_tiling_on_sc=False),
        scratch_types=[
            pltpu.VMEM((NB, KB), jnp.int32),
            pltpu.VMEM((NPP,), jnp.float32),
        ],
    )
    def k(dst_hbm, out_hbm, didx, acc):
        c = lax.axis_index("c")
        s = lax.axis_index("s")
        w = s * NC + c
        pltpu.sync_copy(dst_hbm.at[w], didx)
        def zrow(i, carry):
            acc[pl.ds(i * 16, 16)] = jnp.zeros((16,), jnp.float32)
            return carry
        lax.fori_loop(0, NPP // 16, zrow, None)
        ones = jnp.full((16,), 1.0, jnp.float32)

        def blk(b, carry):
            def sub(j, carry2):
                v = didx[b, pl.ds(j * 16, 16)]
                plsc.addupdate_scatter(acc, [v], ones)
                return carry2
            lax.fori_loop(0, KB // 16, sub, None)
            return carry
        lax.fori_loop(0, NB, blk, None)
        pltpu.sync_copy(acc, out_hbm.at[pl.ds(w * NPP, NPP)])

    return k(dst3)


def _sc_agg(tables, src3, dst3):
    P = len(tables)
    ZR = 392

    @functools.partial(
        pl.kernel,
        out_type=[jax.ShapeDtypeStruct((NC, NP, 16), jnp.float32)] * P,
        mesh=plsc.VectorSubcoreMesh(**_MESH),
        compiler_params=pltpu.CompilerParams(needs_layout_passes=False, use_tc_tiling_on_sc=False),
        scratch_types=[
            pltpu.VMEM((NB, KB), jnp.int32),
            pltpu.VMEM((NB, KB), jnp.int32),
            pltpu.VMEM((2, 8, KB, 16), jnp.float32),
            pltpu.VMEM((ZR, 16), jnp.float32),
            pltpu.VMEM_SHARED((NP, 16), jnp.float32),
            pltpu.SemaphoreType.DMA,
            pltpu.SemaphoreType.DMA,
        ],
    )
    def k(*refs):
        tabs = refs[:P]
        src_hbm, dst_hbm = refs[P], refs[P + 1]
        outs = refs[P + 2:2 * P + 2]
        sidx, didx, rows, zb, sacc, gsem, ssem = refs[2 * P + 2:]
        c = lax.axis_index("c")
        s = lax.axis_index("s")
        w = s * NC + c
        pltpu.sync_copy(src_hbm.at[w], sidx)
        pltpu.sync_copy(dst_hbm.at[w], didx)
        _zero_rows16(zb, ZR)
        for d in range(P):
            def zblk(q, carry):
                pltpu.sync_copy(zb, sacc.at[pl.ds(s * RTN + q * ZR, ZR)])
                return carry
            lax.fori_loop(0, RTN // ZR, zblk, None)
            plsc.subcore_barrier()

            tab = tabs[d]
            G = 8
            NG = NB // G
            def dwait(sem):
                pltpu.make_async_copy(tab.at[sidx.at[0]], rows.at[0, 0],
                                      sem).wait()
            for j in range(G):
                pltpu.async_copy(tab.at[sidx.at[j]], rows.at[0, j], gsem)
            for _ in range(G):
                dwait(gsem)
            for j in range(G):
                pltpu.async_copy(rows.at[0, j], sacc.at[didx.at[j]], ssem,
                                 add=True)
            for j in range(G):
                pltpu.async_copy(tab.at[sidx.at[G + j]], rows.at[1, j], gsem)

            def grp(g, carry):
                p = g & 1
                for _ in range(G):
                    dwait(gsem)
                for j in range(G):
                    pltpu.async_copy(rows.at[p, j],
                                     sacc.at[didx.at[g * G + j]], ssem,
                                     add=True)
                for _ in range(G):
                    dwait(ssem)
                for j in range(G):
                    pltpu.async_copy(tab.at[sidx.at[(g + 1) * G + j]],
                                     rows.at[1 - p, j], gsem)
                return carry
            lax.fori_loop(1, NG - 1, grp, None)

            for _ in range(G):
                dwait(gsem)
            for j in range(G):
                pltpu.async_copy(rows.at[(NG - 1) & 1, j],
                                 sacc.at[didx.at[(NG - 1) * G + j]], ssem,
                                 add=True)
            for _ in range(2 * G):
                dwait(ssem)

            plsc.subcore_barrier()
            pltpu.sync_copy(sacc.at[pl.ds(s * RTN, RTN)],
                            outs[d].at[c, pl.ds(s * RTN, RTN)])

    return k(*tables, src3, dst3)


def _sc_agg1(g3f, src3, dst3):
    CH = 50
    @functools.partial(
        pl.kernel,
        out_type=jax.ShapeDtypeStruct((NWK * NPP,), jnp.float32),
        mesh=plsc.VectorSubcoreMesh(**_MESH),
        compiler_params=pltpu.CompilerParams(needs_layout_passes=False, use_tc_tiling_on_sc=False),
        scratch_types=[
            pltpu.VMEM((NPP,), jnp.float32),
            pltpu.VMEM((NPP,), jnp.float32),
            pltpu.VMEM((CH, KB), jnp.int32),
            pltpu.VMEM((CH, KB), jnp.int32),
        ],
    )
    def k(g3_hbm, src_hbm, dst_hbm, out_hbm, g3v, acc, sidx, didx):
        c = lax.axis_index("c")
        s = lax.axis_index("s")
        w = s * NC + c
        pltpu.sync_copy(g3_hbm, g3v)
        def zrow(i, carry):
            acc[pl.ds(i * 16, 16)] = jnp.zeros((16,), jnp.float32)
            return carry
        lax.fori_loop(0, NPP // 16, zrow, None)

        for q in range(NB // CH):
            pltpu.sync_copy(src_hbm.at[w, pl.ds(q * CH, CH)], sidx)
            pltpu.sync_copy(dst_hbm.at[w, pl.ds(q * CH, CH)], didx)

            def blk(b, carry):
                def sub(j, carry2):
                    sv = sidx[b, pl.ds(j * 16, 16)]
                    dv = didx[b, pl.ds(j * 16, 16)]
                    vals = plsc.load_gather(g3v, [sv])
                    plsc.addupdate_scatter(acc, [dv], vals)
                    return carry2
                lax.fori_loop(0, KB // 16, sub, None)
                return carry
            lax.fori_loop(0, CH, blk, None)

        pltpu.sync_copy(acc, out_hbm.at[pl.ds(w * NPP, NPP)])

    return k(g3f, src3, dst3)


def kernel(x, edge_index, W1, b1, W2, b2, W3, b3):
    src = edge_index[0].astype(jnp.int32)
    dst = edge_index[1].astype(jnp.int32)
    pad = jnp.full((EP - E,), N, jnp.int32)
    src3 = jnp.concatenate([src, pad]).reshape(NWK, NB, KB)
    dst3 = jnp.concatenate([dst, pad]).reshape(NWK, NB, KB)

    xp = jnp.pad(x, ((0, NP - N), (0, F1 - x.shape[1])))
    w1p = jnp.pad(W1, ((0, F1 - W1.shape[0]), (0, 0)))

    degp = _sc_deg(dst3).reshape(NWK, NPP)
    dinv1 = _tc_k1(degp)
    dinv8 = jnp.broadcast_to(dinv1[:NP, None], (NP, 8))
    g1 = _tc_k1g(xp, dinv8)

    s1p = _sc_agg([g1], src3, dst3)[0]
    g2s = _tc_k2(s1p, g1, dinv8, w1p, b1[None, :], W2)

    s2p = _sc_agg(g2s, src3, dst3)
    g3 = _tc_k3(s2p, g2s, dinv8, b2[None, :], W3.reshape(1, MID))[:, 0]

    g3f = jnp.pad(g3, (0, NPP - NP))
    s3p = _sc_agg1(g3f, src3, dst3).reshape(NWK, NPP)
    b3b = jnp.broadcast_to(b3, (NPP,))
    out = _tc_k4(s3p, g3f, dinv1, b3b)
    return out[:N]


# speedup vs baseline: 1.3683x; 1.3683x over previous
import functools

import jax
import jax.numpy as jnp
from jax import lax
from jax.experimental import pallas as pl
from jax.experimental.pallas import tpu as pltpu
from jax.experimental.pallas import tpu_sc as plsc

N = 50000
NP = 50176
RP = NP // 16
E = 800000
EP = 819200
F1 = 16
HID = 256
MID = 128
BN = 1024
BR = BN // 16
GN = NP // BN


def _k1_body(degp_ref, dinv_ref):
    deg = jnp.sum(degp_ref[...], axis=0) + 1.0
    dinv_ref[...] = 1.0 / jnp.sqrt(deg)


def _tc_k1(degp):
    return pl.pallas_call(
        _k1_body,
        grid=(NPP // BN,),
        in_specs=[pl.BlockSpec((NWK, BN), lambda i: (0, i))],
        out_specs=pl.BlockSpec((BN,), lambda i: (i,)),
        out_shape=jax.ShapeDtypeStruct((NPP,), jnp.float32),
    )(degp)


def _k1g_body(x_ref, dinv_ref, g1_ref):
    dflat = jnp.max(dinv_ref[...], axis=1)
    g1_ref[...] = x_ref[...] * dflat[:, None]


def _tc_k1g(x, dinv8):
    return pl.pallas_call(
        _k1g_body,
        grid=(GN,),
        in_specs=[
            pl.BlockSpec((BN, F1), lambda i: (i, 0)),
            pl.BlockSpec((BN, 8), lambda i: (i, 0)),
        ],
        out_specs=pl.BlockSpec((BN, F1), lambda i: (i, 0)),
        out_shape=jax.ShapeDtypeStruct((NP, F1), jnp.float32),
    )(x, dinv8)


def _k2_body(s1_ref, g1_ref, dinv_ref, w1_ref, b1_ref, w2_ref, *g2_refs):
    dflat = jnp.max(dinv_ref[...], axis=1)
    z1 = (s1_ref[0] + s1_ref[1] + g1_ref[...]) * dflat[:, None]
    h1 = jnp.dot(z1, w1_ref[...], preferred_element_type=jnp.float32)
    h1 = jnp.maximum(h1 + b1_ref[...], 0.0)
    m2 = jnp.dot(h1, w2_ref[...], preferred_element_type=jnp.float32)
    g2 = m2 * dflat[:, None]
    for d in range(8):
        g2_refs[d][...] = g2[:, d * 16:(d + 1) * 16]


def _tc_k2(s1p, g1, dinv8, w1p, b1, w2):
    return pl.pallas_call(
        _k2_body,
        grid=(GN,),
        in_specs=[
            pl.BlockSpec((2, BN, F1), lambda i: (0, i, 0)),
            pl.BlockSpec((BN, F1), lambda i: (i, 0)),
            pl.BlockSpec((BN, 8), lambda i: (i, 0)),
            pl.BlockSpec((F1, HID), lambda i: (0, 0)),
            pl.BlockSpec((1, HID), lambda i: (0, 0)),
            pl.BlockSpec((HID, MID), lambda i: (0, 0)),
        ],
        out_specs=[pl.BlockSpec((BN, 16), lambda i: (i, 0))] * 8,
        out_shape=[jax.ShapeDtypeStruct((NP, 16), jnp.float32)] * 8,
    )(s1p, g1, dinv8, w1p, b1, w2)


def _k3_body(*refs):
    srefs = refs[0:8]
    grefs = refs[8:16]
    dinv_ref, b2_ref, w3_ref, g3_ref = refs[16:]
    dflat = jnp.max(dinv_ref[...], axis=1)
    cols = [srefs[d][0] + srefs[d][1] + grefs[d][...] for d in range(8)]
    z2 = jnp.concatenate(cols, axis=1) * dflat[:, None]
    h2 = jnp.maximum(z2 + b2_ref[...], 0.0)
    m3 = jnp.sum(h2 * w3_ref[...], axis=1)
    g3_ref[...] = (m3 * dflat)[:, None] * jnp.ones((1, 8), jnp.float32)


def _tc_k3(s2p, g2s, dinv8, b2, w3):
    return pl.pallas_call(
        _k3_body,
        grid=(GN,),
        in_specs=(
            [pl.BlockSpec((2, BN, 16), lambda i: (0, i, 0))] * 8
            + [pl.BlockSpec((BN, 16), lambda i: (i, 0))] * 8
            + [
                pl.BlockSpec((BN, 8), lambda i: (i, 0)),
                pl.BlockSpec((1, MID), lambda i: (0, 0)),
                pl.BlockSpec((1, MID), lambda i: (0, 0)),
            ]
        ),
        out_specs=pl.BlockSpec((BN, 8), lambda i: (i, 0)),
        out_shape=jax.ShapeDtypeStruct((NP, 8), jnp.float32),
    )(*s2p, *g2s, dinv8, b2, w3)


def _k4_body(s3_ref, g3_ref, dinv_ref, b3_ref, out_ref):
    s3 = jnp.sum(s3_ref[...], axis=0)
    out_ref[...] = (s3 + g3_ref[...]) * dinv_ref[...] + b3_ref[...]


def _tc_k4(s3p, g3f, dinv1, b3b):
    return pl.pallas_call(
        _k4_body,
        grid=(NPP // BN,),
        in_specs=[
            pl.BlockSpec((NWK, BN), lambda i: (0, i)),
            pl.BlockSpec((BN,), lambda i: (i,)),
            pl.BlockSpec((BN,), lambda i: (i,)),
            pl.BlockSpec((BN,), lambda i: (i,)),
        ],
        out_specs=pl.BlockSpec((BN,), lambda i: (i,)),
        out_shape=jax.ShapeDtypeStruct((NPP,), jnp.float32),
    )(s3p, g3f, dinv1, b3b)


NC, NS, NWK = 2, 16, 32
NB, KB = 400, 64
RPP = 3200
NPP = RPP * 16
RT = RPP // NS
RTN = NP // NS
IDR, IDC = 25, 128
NBA, NBB = 480, 320

_MESH = dict(core_axis_name="c", subcore_axis_name="s")


def _zero_rows16(ref, n):
    def body(i, carry):
        ref[i] = jnp.zeros((16,), jnp.float32)
        return carry
    lax.fori_loop(0, n, body, None)


def _zero_rows32(ref, n):
    def body(i, carry):
        ref[i, pl.ds(0, 16)] = jnp.zeros((16,), jnp.float32)
        ref[i, pl.ds(16, 16)] = jnp.zeros((16,), jnp.float32)
        return carry
    lax.fori_loop(0, n, body, None)


def _sc_deg(dst3):
    @functools.partial(
        pl.kernel,
        out_type=jax.ShapeDtypeStruct((NWK * NPP,), jnp.float32),
        mesh=plsc.VectorSubcoreMesh(**_MESH),
        compiler_params=pltpu.CompilerParams(needs_layout_passes=False, use_tc_tiling_on_sc=False),
        scratch_types=[
            pltpu.VMEM((NB, KB), jnp.int32),
            pltpu.VMEM((NPP,), jnp.float32),
        ],
    )
    def k(dst_hbm, out_hbm, didx, acc):
        c = lax.axis_index("c")
        s = lax.axis_index("s")
        w = s * NC + c
        pltpu.sync_copy(dst_hbm.at[w], didx)
        def zrow(i, carry):
            acc[pl.ds(i * 16, 16)] = jnp.zeros((16,), jnp.float32)
            return carry
        lax.fori_loop(0, NPP // 16, zrow, None)
        ones = jnp.full((16,), 1.0, jnp.float32)

        def blk(b, carry):
            def sub(j, carry2):
                v = didx[b, pl.ds(j * 16, 16)]
                plsc.addupdate_scatter(acc, [v], ones)
                return carry2
            lax.fori_loop(0, KB // 16, sub, None)
            return carry
        lax.fori_loop(0, NB, blk, None)
        pltpu.sync_copy(acc, out_hbm.at[pl.ds(w * NPP, NPP)])

    return k(dst3)


def _sc_agg(tables, src2, dst2):
    P = len(tables)
    ZR = 392

    @functools.partial(
        pl.kernel,
        out_type=[jax.ShapeDtypeStruct((NC, NP, 16), jnp.float32)] * P,
        mesh=plsc.VectorSubcoreMesh(**_MESH),
        compiler_params=pltpu.CompilerParams(needs_layout_passes=False, use_tc_tiling_on_sc=False),
        scratch_types=[
            pltpu.VMEM((NBA, KB), jnp.int32),
            pltpu.VMEM((NBA, KB), jnp.int32),
            pltpu.VMEM((2, 4, KB, 16), jnp.float32),
            pltpu.VMEM((ZR, 16), jnp.float32),
            pltpu.VMEM_SHARED((NP, 16), jnp.float32),
            pltpu.SemaphoreType.DMA,
            pltpu.SemaphoreType.DMA,
        ],
    )
    def k(*refs):
        tabs = refs[:P]
        src_hbm, dst_hbm = refs[P], refs[P + 1]
        outs = refs[P + 2:2 * P + 2]
        sidx, didx, rows, zb, sacc, gsem, ssem = refs[2 * P + 2:]
        c = lax.axis_index("c")
        s = lax.axis_index("s")
        base = jnp.where(c == 0, s * NBA, NS * NBA + s * NBB)
        ngd = jnp.where(c == 0, NBA // 8, NBB // 8)
        pltpu.sync_copy(src_hbm.at[pl.ds(base, NBA)], sidx)
        pltpu.sync_copy(dst_hbm.at[pl.ds(base, NBA)], didx)
        _zero_rows16(zb, ZR)
        for d in range(P):
            def zblk(q, carry):
                pltpu.sync_copy(zb, sacc.at[pl.ds(s * RTN + q * ZR, ZR)])
                return carry
            lax.fori_loop(0, RTN // ZR, zblk, None)
            plsc.subcore_barrier()

            tab = tabs[d]
            G = 4
            def dwait(sem):
                pltpu.make_async_copy(tab.at[sidx.at[0]], rows.at[0, 0],
                                      sem).wait()
            for j in range(G):
                pltpu.async_copy(tab.at[sidx.at[j]], rows.at[0, j], gsem)
            for _ in range(G):
                dwait(gsem)
            for j in range(G):
                pltpu.async_copy(rows.at[0, j], sacc.at[didx.at[j]], ssem,
                                 add=True)
            for j in range(G):
                pltpu.async_copy(tab.at[sidx.at[G + j]], rows.at[1, j], gsem)

            def grp(g, carry):
                p = g & 1
                for _ in range(G):
                    dwait(gsem)
                for j in range(G):
                    pltpu.async_copy(rows.at[p, j],
                                     sacc.at[didx.at[g * G + j]], ssem,
                                     add=True)
                for _ in range(G):
                    dwait(ssem)
                for j in range(G):
                    pltpu.async_copy(tab.at[sidx.at[(g + 1) * G + j]],
                                     rows.at[1 - p, j], gsem)
                return carry
            lax.fori_loop(1, ngd - 1, grp, None)

            gl = ngd - 1
            for _ in range(G):
                dwait(gsem)
            for j in range(G):
                pltpu.async_copy(rows.at[gl & 1, j],
                                 sacc.at[didx.at[gl * G + j]], ssem,
                                 add=True)
            for _ in range(2 * G):
                dwait(ssem)

            plsc.subcore_barrier()
            pltpu.sync_copy(sacc.at[pl.ds(s * RTN, RTN)],
                            outs[d].at[c, pl.ds(s * RTN, RTN)])

    return k(*tables, src2, dst2)


def _sc_agg1(g3f, src3, dst3):
    CH = 50
    @functools.partial(
        pl.kernel,
        out_type=jax.ShapeDtypeStruct((NWK * NPP,), jnp.float32),
        mesh=plsc.VectorSubcoreMesh(**_MESH),
        compiler_params=pltpu.CompilerParams(needs_layout_passes=False, use_tc_tiling_on_sc=False),
        scratch_types=[
            pltpu.VMEM((NPP,), jnp.float32),
            pltpu.VMEM((NPP,), jnp.float32),
            pltpu.VMEM((CH, KB), jnp.int32),
            pltpu.VMEM((CH, KB), jnp.int32),
        ],
    )
    def k(g3_hbm, src_hbm, dst_hbm, out_hbm, g3v, acc, sidx, didx):
        c = lax.axis_index("c")
        s = lax.axis_index("s")
        w = s * NC + c
        pltpu.sync_copy(g3_hbm, g3v)
        def zrow(i, carry):
            acc[pl.ds(i * 16, 16)] = jnp.zeros((16,), jnp.float32)
            return carry
        lax.fori_loop(0, NPP // 16, zrow, None)

        for q in range(NB // CH):
            pltpu.sync_copy(src_hbm.at[w, pl.ds(q * CH, CH)], sidx)
            pltpu.sync_copy(dst_hbm.at[w, pl.ds(q * CH, CH)], didx)

            def blk(b, carry):
                def sub(j, carry2):
                    sv = sidx[b, pl.ds(j * 16, 16)]
                    dv = didx[b, pl.ds(j * 16, 16)]
                    vals = plsc.load_gather(g3v, [sv])
                    plsc.addupdate_scatter(acc, [dv], vals)
                    return carry2
                lax.fori_loop(0, KB // 16, sub, None)
                return carry
            lax.fori_loop(0, CH, blk, None)

        pltpu.sync_copy(acc, out_hbm.at[pl.ds(w * NPP, NPP)])

    return k(g3f, src3, dst3)


def kernel(x, edge_index, W1, b1, W2, b2, W3, b3):
    src = edge_index[0].astype(jnp.int32)
    dst = edge_index[1].astype(jnp.int32)
    pad = jnp.full((EP - E,), N, jnp.int32)
    src3 = jnp.concatenate([src, pad]).reshape(NWK, NB, KB)
    dst3 = jnp.concatenate([dst, pad]).reshape(NWK, NB, KB)
    ext = NS * NBA + (NS - 1) * NBB + NBA - NWK * NB
    src2 = jnp.pad(src3.reshape(NWK * NB, KB), ((0, ext), (0, 0)))
    dst2 = jnp.pad(dst3.reshape(NWK * NB, KB), ((0, ext), (0, 0)))

    xp = jnp.pad(x, ((0, NP - N), (0, F1 - x.shape[1])))
    w1p = jnp.pad(W1, ((0, F1 - W1.shape[0]), (0, 0)))

    degp = _sc_deg(dst3).reshape(NWK, NPP)
    dinv1 = _tc_k1(degp)
    dinv8 = jnp.broadcast_to(dinv1[:NP, None], (NP, 8))
    g1 = _tc_k1g(xp, dinv8)

    s1p = _sc_agg([g1], src2, dst2)[0]
    g2s = _tc_k2(s1p, g1, dinv8, w1p, b1[None, :], W2)

    s2p = _sc_agg(g2s, src2, dst2)
    g3 = _tc_k3(s2p, g2s, dinv8, b2[None, :], W3.reshape(1, MID))[:, 0]

    g3f = jnp.pad(g3, (0, NPP - NP))
    s3p = _sc_agg1(g3f, src3, dst3).reshape(NWK, NPP)
    b3b = jnp.broadcast_to(b3, (NPP,))
    out = _tc_k4(s3p, g3f, dinv1, b3b)
    return out[:N]
